# baseline (speedup 1.0000x reference)
import functools

import jax
import jax.numpy as jnp
from jax import lax
from jax.experimental import pallas as pl
from jax.experimental.pallas import tpu as pltpu
from jax.experimental.pallas import tpu_sc as plsc

B = 16384
V = 100000
D = 16
F = 26

NC = 2
NS = 16
NW = NC * NS
BPW = B // NW
CH = 512
NCH = BPW // CH

RG = 8192
NG = -(-V // RG)
VP = NG * RG

_SC_PARAMS = dict(
    compiler_params=pltpu.CompilerParams(
        needs_layout_passes=False, use_tc_tiling_on_sc=False),
)


def _prep_body(bias_ref, eT_ref, g_ref):
    e = eT_ref[...]
    norm2 = jnp.sum(e * e, axis=0)
    g_ref[...] = bias_ref[0] * (1.0 / F) - 0.5 * norm2


def _prep_tables(embed_table, w_table, bias):
    g0 = pl.pallas_call(
        _prep_body,
        grid=(NG,),
        in_specs=[
            pl.BlockSpec(memory_space=pltpu.SMEM),
            pl.BlockSpec((D, RG), lambda i: (0, i)),
        ],
        out_specs=pl.BlockSpec((RG,), lambda i: (i,)),
        out_shape=jax.ShapeDtypeStruct((VP,), jnp.float32),
    )(bias, embed_table.T)
    w_flat = jnp.pad(w_table.T.reshape(V), (0, VP - V))
    return g0 + w_flat


def _worker_base():
    cid = lax.axis_index("c")
    sid = lax.axis_index("s")
    wid = sid * NC + cid
    return wid, wid * BPW


def _g_sc_body(x_hbm, g_hbm, zg_hbm, idx_v, gacc_v, sg_a, sg_b, sem_a, sem_b):
    wid, base = _worker_base()
    pltpu.sync_copy(x_hbm.at[wid], idx_v)

    def fire(f, dst, sem):
        for c in range(NCH):
            idx = idx_v.at[f * NCH + c]
            pltpu.async_copy(g_hbm.at[idx], dst.at[pl.ds(c * CH, CH)], sem)

    def drain(dst, sem):
        for c in range(NCH):
            pltpu.make_async_copy(
                g_hbm.at[idx_v.at[0]], dst.at[pl.ds(c * CH, CH)], sem).wait()

    def accum(src):
        def gadd(i, c2):
            j = i * 64
            for u in range(0, 64, 16):
                plsc.addupdate(gacc_v.at[pl.ds(j + u, 16)],
                               src[pl.ds(j + u, 16)])
            return c2

        lax.fori_loop(0, BPW // 64, gadd, 0)

    fire(0, gacc_v, sem_b)
    fire(1, sg_a, sem_a)
    drain(gacc_v, sem_b)

    def floop2(j, carry):
        fb = 2 * j + 2
        fire(fb, sg_b, sem_b)
        drain(sg_a, sem_a)
        accum(sg_a)
        fire(fb + 1, sg_a, sem_a)
        drain(sg_b, sem_b)
        accum(sg_b)
        return carry

    lax.fori_loop(0, (F - 2) // 2, floop2, 0)
    drain(sg_a, sem_a)
    accum(sg_a)

    pltpu.sync_copy(gacc_v, zg_hbm.at[pl.ds(base, BPW)])


def _e_sc_body(x_hbm, emb_hbm, zg_hbm, out_hbm,
               idx_v, acc_v, zg_v, se_a, se_b, sq_v, o_v, sem_a, sem_b):
    wid, base = _worker_base()
    pltpu.sync_copy(x_hbm.at[wid], idx_v)
    pltpu.sync_copy(zg_hbm.at[pl.ds(base, BPW)], zg_v)

    def fire(f, dst, sem):
        for c in range(NCH):
            idx = idx_v.at[f * NCH + c]
            pltpu.async_copy(emb_hbm.at[idx], dst.at[pl.ds(c * CH, CH)], sem)

    def drain(dst, sem):
        for c in range(NCH):
            pltpu.make_async_copy(
                emb_hbm.at[idx_v.at[0]], dst.at[pl.ds(c * CH, CH)], sem).wait()

    def accum(src):
        def rowadd(i, c2):
            b = i * 16
            for u in range(16):
                plsc.addupdate(acc_v.at[b + u], src[b + u, :])
            return c2

        lax.fori_loop(0, BPW // 16, rowadd, 0)

    fire(0, acc_v, sem_b)
    fire(1, se_a, sem_a)
    drain(acc_v, sem_b)

    def floop2(j, carry):
        fb = 2 * j + 2
        fire(fb, se_b, sem_b)
        drain(se_a, sem_a)
        accum(se_a)
        fire(fb + 1, se_a, sem_a)
        drain(se_b, sem_b)
        accum(se_b)
        return carry

    lax.fori_loop(0, (F - 2) // 2, floop2, 0)
    drain(se_a, sem_a)
    accum(se_a)

    lane_off = lax.iota(jnp.int32, 16) * BPW

    def rowloop(i, carry):
        b = i * 8
        for u in range(8):
            v = acc_v[b + u, :]
            plsc.store_scatter(sq_v, [lane_off + (b + u)], v * v)
        return carry

    lax.fori_loop(0, BPW // 8, rowloop, 0)

    def sigloop(j, carry):
        s = sq_v[pl.ds(j * 16, 16)]
        for d in range(1, D):
            s = s + sq_v[pl.ds(d * BPW + j * 16, 16)]
        z = zg_v[pl.ds(j * 16, 16)] + 0.5 * s
        o_v[pl.ds(j * 16, 16)] = 1.0 / (1.0 + jnp.exp(-z))
        return carry

    lax.fori_loop(0, BPW // 16, sigloop, 0)

    pltpu.sync_copy(o_v, out_hbm.at[pl.ds(base, BPW)])


@jax.jit
def _fm_forward(embed_table, w_table, bias, xs):
    g_table = _prep_tables(embed_table, w_table, bias)

    x_arr = xs.reshape(F, NW, NCH, CH).transpose(1, 0, 2, 3).reshape(NW, F * NCH, CH)

    mesh = plsc.VectorSubcoreMesh(
        core_axis_name="c", subcore_axis_name="s", num_cores=NC, num_subcores=NS)

    zg = pl.kernel(
        _g_sc_body,
        out_type=jax.ShapeDtypeStruct((B,), jnp.float32),
        mesh=mesh,
        scratch_types=[
            pltpu.VMEM((F * NCH, CH), jnp.int32),
            pltpu.VMEM((BPW,), jnp.float32),
            pltpu.VMEM((BPW,), jnp.float32),
            pltpu.VMEM((BPW,), jnp.float32),
            pltpu.SemaphoreType.DMA,
            pltpu.SemaphoreType.DMA,
        ],
        **_SC_PARAMS,
    )(x_arr, g_table)

    out = pl.kernel(
        _e_sc_body,
        out_type=jax.ShapeDtypeStruct((B,), jnp.float32),
        mesh=mesh,
        scratch_types=[
            pltpu.VMEM((F * NCH, CH), jnp.int32),
            pltpu.VMEM((BPW, D), jnp.float32),
            pltpu.VMEM((BPW,), jnp.float32),
            pltpu.VMEM((BPW, D), jnp.float32),
            pltpu.VMEM((BPW, D), jnp.float32),
            pltpu.VMEM((D * BPW,), jnp.float32),
            pltpu.VMEM((BPW,), jnp.float32),
            pltpu.SemaphoreType.DMA,
            pltpu.SemaphoreType.DMA,
        ],
        **_SC_PARAMS,
    )(x_arr, embed_table, zg)
    return out.reshape(B, 1)


def kernel(embed_table, w_table, bias,
           f0, f1, f2, f3, f4, f5, f6, f7, f8, f9,
           f10, f11, f12, f13, f14, f15, f16, f17, f18, f19,
           f20, f21, f22, f23, f24, f25):
    xs = jnp.stack([f0, f1, f2, f3, f4, f5, f6, f7, f8, f9,
                    f10, f11, f12, f13, f14, f15, f16, f17, f18, f19,
                    f20, f21, f22, f23, f24, f25], axis=0)
    return _fm_forward(embed_table, w_table, bias, xs)


# speedup vs baseline: 1.0308x; 1.0308x over previous
import functools

import jax
import jax.numpy as jnp
from jax import lax
from jax.experimental import pallas as pl
from jax.experimental.pallas import tpu as pltpu
from jax.experimental.pallas import tpu_sc as plsc

B = 16384
V = 100000
D = 16
F = 26

NC = 2
NS = 16
NW = NC * NS
BPW = B // NW
CH = 512
NCH = BPW // CH

RG = 16384
NG = -(-V // RG)
VP = NG * RG

_SC_PARAMS = dict(
    compiler_params=pltpu.CompilerParams(
        needs_layout_passes=False, use_tc_tiling_on_sc=False),
)


def _prep_body(bias_ref, eT_ref, g_ref):
    e = eT_ref[...]
    norm2 = jnp.sum(e * e, axis=0)
    g_ref[...] = bias_ref[0] * (1.0 / F) - 0.5 * norm2


def _prep_tables(embed_table, w_table, bias):
    g0 = pl.pallas_call(
        _prep_body,
        grid=(NG,),
        in_specs=[
            pl.BlockSpec(memory_space=pltpu.SMEM),
            pl.BlockSpec((D, RG), lambda i: (0, i)),
        ],
        out_specs=pl.BlockSpec((RG,), lambda i: (i,)),
        out_shape=jax.ShapeDtypeStruct((VP,), jnp.float32),
    )(bias, embed_table.T)
    w_flat = jnp.pad(w_table.T.reshape(V), (0, VP - V))
    return g0 + w_flat


def _worker_base():
    cid = lax.axis_index("c")
    sid = lax.axis_index("s")
    wid = sid * NC + cid
    return wid, wid * BPW


def _g_sc_body(x_hbm, g_hbm, zg_hbm, idx_v, gacc_v, sg_a, sg_b, sem_a, sem_b):
    wid, base = _worker_base()
    pltpu.sync_copy(x_hbm.at[wid], idx_v)

    def fire(f, dst, sem):
        for c in range(NCH):
            idx = idx_v.at[f * NCH + c]
            pltpu.async_copy(g_hbm.at[idx], dst.at[pl.ds(c * CH, CH)], sem)

    def drain(dst, sem):
        for c in range(NCH):
            pltpu.make_async_copy(
                g_hbm.at[idx_v.at[0]], dst.at[pl.ds(c * CH, CH)], sem).wait()

    def accum(src):
        def gadd(i, c2):
            j = i * 64
            for u in range(0, 64, 16):
                plsc.addupdate(gacc_v.at[pl.ds(j + u, 16)],
                               src[pl.ds(j + u, 16)])
            return c2

        lax.fori_loop(0, BPW // 64, gadd, 0)

    fire(0, gacc_v, sem_b)
    fire(1, sg_a, sem_a)
    drain(gacc_v, sem_b)

    def floop2(j, carry):
        fb = 2 * j + 2
        fire(fb, sg_b, sem_b)
        drain(sg_a, sem_a)
        accum(sg_a)
        fire(fb + 1, sg_a, sem_a)
        drain(sg_b, sem_b)
        accum(sg_b)
        return carry

    lax.fori_loop(0, (F - 2) // 2, floop2, 0)
    drain(sg_a, sem_a)
    accum(sg_a)

    pltpu.sync_copy(gacc_v, zg_hbm.at[pl.ds(base, BPW)])


def _e_sc_body(x_hbm, emb_hbm, zg_hbm, out_hbm,
               idx_v, acc_v, zg_v, se_a, se_b, sq_v, o_v, sem_a, sem_b):
    wid, base = _worker_base()
    pltpu.sync_copy(x_hbm.at[wid], idx_v)
    pltpu.sync_copy(zg_hbm.at[pl.ds(base, BPW)], zg_v)

    def fire(f, dst, sem):
        for c in range(NCH):
            idx = idx_v.at[f * NCH + c]
            pltpu.async_copy(emb_hbm.at[idx], dst.at[pl.ds(c * CH, CH)], sem)

    def drain(dst, sem):
        for c in range(NCH):
            pltpu.make_async_copy(
                emb_hbm.at[idx_v.at[0]], dst.at[pl.ds(c * CH, CH)], sem).wait()

    def accum(src):
        def rowadd(i, c2):
            b = i * 16
            for u in range(16):
                plsc.addupdate(acc_v.at[b + u], src[b + u, :])
            return c2

        lax.fori_loop(0, BPW // 16, rowadd, 0)

    fire(0, acc_v, sem_b)
    fire(1, se_a, sem_a)
    drain(acc_v, sem_b)

    def floop2(j, carry):
        fb = 2 * j + 2
        fire(fb, se_b, sem_b)
        drain(se_a, sem_a)
        accum(se_a)
        fire(fb + 1, se_a, sem_a)
        drain(se_b, sem_b)
        accum(se_b)
        return carry

    lax.fori_loop(0, (F - 2) // 2, floop2, 0)
    drain(se_a, sem_a)
    accum(se_a)

    lane_off = lax.iota(jnp.int32, 16) * BPW

    def rowloop(i, carry):
        b = i * 8
        for u in range(8):
            v = acc_v[b + u, :]
            plsc.store_scatter(sq_v, [lane_off + (b + u)], v * v)
        return carry

    lax.fori_loop(0, BPW // 8, rowloop, 0)

    def sigloop(j, carry):
        s = sq_v[pl.ds(j * 16, 16)]
        for d in range(1, D):
            s = s + sq_v[pl.ds(d * BPW + j * 16, 16)]
        z = zg_v[pl.ds(j * 16, 16)] + 0.5 * s
        o_v[pl.ds(j * 16, 16)] = 1.0 / (1.0 + jnp.exp(-z))
        return carry

    lax.fori_loop(0, BPW // 16, sigloop, 0)

    pltpu.sync_copy(o_v, out_hbm.at[pl.ds(base, BPW)])


@jax.jit
def _fm_forward(embed_table, w_table, bias, xs):
    g_table = _prep_tables(embed_table, w_table, bias)

    x_arr = xs.reshape(F, NW, NCH, CH).transpose(1, 0, 2, 3).reshape(NW, F * NCH, CH)

    mesh = plsc.VectorSubcoreMesh(
        core_axis_name="c", subcore_axis_name="s", num_cores=NC, num_subcores=NS)

    zg = pl.kernel(
        _g_sc_body,
        out_type=jax.ShapeDtypeStruct((B,), jnp.float32),
        mesh=mesh,
        scratch_types=[
            pltpu.VMEM((F * NCH, CH), jnp.int32),
            pltpu.VMEM((BPW,), jnp.float32),
            pltpu.VMEM((BPW,), jnp.float32),
            pltpu.VMEM((BPW,), jnp.float32),
            pltpu.SemaphoreType.DMA,
            pltpu.SemaphoreType.DMA,
        ],
        **_SC_PARAMS,
    )(x_arr, g_table)

    out = pl.kernel(
        _e_sc_body,
        out_type=jax.ShapeDtypeStruct((B,), jnp.float32),
        mesh=mesh,
        scratch_types=[
            pltpu.VMEM((F * NCH, CH), jnp.int32),
            pltpu.VMEM((BPW, D), jnp.float32),
            pltpu.VMEM((BPW,), jnp.float32),
            pltpu.VMEM((BPW, D), jnp.float32),
            pltpu.VMEM((BPW, D), jnp.float32),
            pltpu.VMEM((D * BPW,), jnp.float32),
            pltpu.VMEM((BPW,), jnp.float32),
            pltpu.SemaphoreType.DMA,
            pltpu.SemaphoreType.DMA,
        ],
        **_SC_PARAMS,
    )(x_arr, embed_table, zg)
    return out.reshape(B, 1)


def kernel(embed_table, w_table, bias,
           f0, f1, f2, f3, f4, f5, f6, f7, f8, f9,
           f10, f11, f12, f13, f14, f15, f16, f17, f18, f19,
           f20, f21, f22, f23, f24, f25):
    xs = jnp.stack([f0, f1, f2, f3, f4, f5, f6, f7, f8, f9,
                    f10, f11, f12, f13, f14, f15, f16, f17, f18, f19,
                    f20, f21, f22, f23, f24, f25], axis=0)
    return _fm_forward(embed_table, w_table, bias, xs)


# speedup vs baseline: 1.0314x; 1.0006x over previous
import jax
import jax.numpy as jnp
from jax import lax
from jax.experimental import pallas as pl
from jax.experimental.pallas import tpu as pltpu
from jax.experimental.pallas import tpu_sc as plsc

B = 16384
V = 100000
D = 16
F = 26

NC = 2
NS = 16
NW = NC * NS
BPW = B // NW
CH = 512
NCH = BPW // CH

RG = 16384
NG = -(-V // RG)
VP = NG * RG

_SC_PARAMS = dict(
    compiler_params=pltpu.CompilerParams(
        needs_layout_passes=False, use_tc_tiling_on_sc=False),
)


def _prep_body(bias_ref, eT_ref, g_ref):
    e = eT_ref[...]
    norm2 = jnp.sum(e * e, axis=0)
    g_ref[...] = bias_ref[0] * (1.0 / F) - 0.5 * norm2


def _prep_tables(embed_table, w_table, bias):
    g0 = pl.pallas_call(
        _prep_body,
        grid=(NG,),
        in_specs=[
            pl.BlockSpec(memory_space=pltpu.SMEM),
            pl.BlockSpec((D, RG), lambda i: (0, i)),
        ],
        out_specs=pl.BlockSpec((RG,), lambda i: (i,)),
        out_shape=jax.ShapeDtypeStruct((VP,), jnp.float32),
    )(bias, embed_table.T)
    w_flat = jnp.pad(w_table.T.reshape(V), (0, VP - V))
    return g0 + w_flat


def _worker_base():
    cid = lax.axis_index("c")
    sid = lax.axis_index("s")
    wid = sid * NC + cid
    return wid, wid * BPW


def _g_sc_body(x_hbm, g_hbm, zg_hbm, idx_v, gacc_v, sg_a, sg_b, sem_a, sem_b):
    wid, base = _worker_base()
    pltpu.sync_copy(x_hbm.at[wid], idx_v)

    def fire(f, dst, sem):
        for c in range(NCH):
            idx = idx_v.at[f * NCH + c]
            pltpu.async_copy(g_hbm.at[idx], dst.at[pl.ds(c * CH, CH)], sem)

    def drain(dst, sem):
        for c in range(NCH):
            pltpu.make_async_copy(
                g_hbm.at[idx_v.at[0]], dst.at[pl.ds(c * CH, CH)], sem).wait()

    def accum(src):
        def gadd(i, c2):
            j = i * 64
            for u in range(0, 64, 16):
                plsc.addupdate(gacc_v.at[pl.ds(j + u, 16)],
                               src[pl.ds(j + u, 16)])
            return c2

        lax.fori_loop(0, BPW // 64, gadd, 0)

    fire(0, gacc_v, sem_b)
    fire(1, sg_a, sem_a)
    drain(gacc_v, sem_b)

    def floop2(j, carry):
        fb = 2 * j + 2
        fire(fb, sg_b, sem_b)
        drain(sg_a, sem_a)
        accum(sg_a)
        fire(fb + 1, sg_a, sem_a)
        drain(sg_b, sem_b)
        accum(sg_b)
        return carry

    lax.fori_loop(0, (F - 2) // 2, floop2, 0)
    drain(sg_a, sem_a)
    accum(sg_a)

    pltpu.sync_copy(gacc_v, zg_hbm.at[pl.ds(base, BPW)])


def _e_sc_body(x_hbm, emb_hbm, zg_hbm, out_hbm,
               idx_v, acc_v, zg_v, se_a, se_b, sq_v, o_v, sem_a, sem_b):
    wid, base = _worker_base()
    pltpu.sync_copy(x_hbm.at[wid], idx_v)
    pltpu.sync_copy(zg_hbm.at[pl.ds(base, BPW)], zg_v)

    def fire(f, dst, sem):
        for c in range(NCH):
            idx = idx_v.at[f * NCH + c]
            pltpu.async_copy(emb_hbm.at[idx], dst.at[pl.ds(c * CH, CH)], sem)

    def drain(dst, sem):
        for c in range(NCH):
            pltpu.make_async_copy(
                emb_hbm.at[idx_v.at[0]], dst.at[pl.ds(c * CH, CH)], sem).wait()

    def accum(src):
        def rowadd(i, c2):
            b = i * 16
            for u in range(16):
                plsc.addupdate(acc_v.at[b + u], src[b + u, :])
            return c2

        lax.fori_loop(0, BPW // 16, rowadd, 0)

    fire(0, acc_v, sem_b)
    fire(1, se_a, sem_a)
    drain(acc_v, sem_b)

    def floop2(j, carry):
        fb = 2 * j + 2
        fire(fb, se_b, sem_b)
        drain(se_a, sem_a)
        accum(se_a)
        fire(fb + 1, se_a, sem_a)
        drain(se_b, sem_b)
        accum(se_b)
        return carry

    lax.fori_loop(0, (F - 2) // 2, floop2, 0)
    drain(se_a, sem_a)
    accum(se_a)

    lane_off = lax.iota(jnp.int32, 16) * BPW

    def rowloop(i, carry):
        b = i * 8
        for u in range(8):
            v = acc_v[b + u, :]
            plsc.store_scatter(sq_v, [lane_off + (b + u)], v * v)
        return carry

    lax.fori_loop(0, BPW // 8, rowloop, 0)

    def sigloop(j, carry):
        s = sq_v[pl.ds(j * 16, 16)]
        for d in range(1, D):
            s = s + sq_v[pl.ds(d * BPW + j * 16, 16)]
        z = zg_v[pl.ds(j * 16, 16)] + 0.5 * s
        o_v[pl.ds(j * 16, 16)] = 1.0 / (1.0 + jnp.exp(-z))
        return carry

    lax.fori_loop(0, BPW // 16, sigloop, 0)

    pltpu.sync_copy(o_v, out_hbm.at[pl.ds(base, BPW)])


@jax.jit
def _fm_forward(embed_table, w_table, bias, xs):
    g_table = _prep_tables(embed_table, w_table, bias)

    x_arr = xs.reshape(F, NW, NCH, CH).transpose(1, 0, 2, 3).reshape(NW, F * NCH, CH)

    mesh = plsc.VectorSubcoreMesh(
        core_axis_name="c", subcore_axis_name="s", num_cores=NC, num_subcores=NS)

    zg = pl.kernel(
        _g_sc_body,
        out_type=jax.ShapeDtypeStruct((B,), jnp.float32),
        mesh=mesh,
        scratch_types=[
            pltpu.VMEM((F * NCH, CH), jnp.int32),
            pltpu.VMEM((BPW,), jnp.float32),
            pltpu.VMEM((BPW,), jnp.float32),
            pltpu.VMEM((BPW,), jnp.float32),
            pltpu.SemaphoreType.DMA,
            pltpu.SemaphoreType.DMA,
        ],
        **_SC_PARAMS,
    )(x_arr, g_table)

    out = pl.kernel(
        _e_sc_body,
        out_type=jax.ShapeDtypeStruct((B,), jnp.float32),
        mesh=mesh,
        scratch_types=[
            pltpu.VMEM((F * NCH, CH), jnp.int32),
            pltpu.VMEM((BPW, D), jnp.float32),
            pltpu.VMEM((BPW,), jnp.float32),
            pltpu.VMEM((BPW, D), jnp.float32),
            pltpu.VMEM((BPW, D), jnp.float32),
            pltpu.VMEM((D * BPW,), jnp.float32),
            pltpu.VMEM((BPW,), jnp.float32),
            pltpu.SemaphoreType.DMA,
            pltpu.SemaphoreType.DMA,
        ],
        **_SC_PARAMS,
    )(x_arr, embed_table, zg)
    return out.reshape(B, 1)


def kernel(embed_table, w_table, bias,
           f0, f1, f2, f3, f4, f5, f6, f7, f8, f9,
           f10, f11, f12, f13, f14, f15, f16, f17, f18, f19,
           f20, f21, f22, f23, f24, f25):
    xs = jnp.stack([f0, f1, f2, f3, f4, f5, f6, f7, f8, f9,
                    f10, f11, f12, f13, f14, f15, f16, f17, f18, f19,
                    f20, f21, f22, f23, f24, f25], axis=0)
    return _fm_forward(embed_table, w_table, bias, xs)
